# baseline (speedup 1.0000x reference)
import functools

import jax
import jax.numpy as jnp
from jax import lax
from jax.experimental import pallas as pl
from jax.experimental.pallas import tpu as pltpu
from jax.experimental.pallas import tpu_sc as plsc

BATCH = 16384
D = 64
NC, NS = 2, 16
NW = NC * NS
BPW = BATCH // NW
CHUNK = 128
NCHUNK = BPW // CHUNK


def _sc_gather3(idx_a, idx_b, idx_r, ent, rel):
    mesh = plsc.VectorSubcoreMesh(core_axis_name="c", subcore_axis_name="s")
    row_ty = jax.ShapeDtypeStruct((BATCH, D), jnp.float32)

    @functools.partial(
        pl.kernel,
        out_type=(row_ty,) * 3,
        mesh=mesh,
        compiler_params=pltpu.CompilerParams(use_tc_tiling_on_sc=False),
        scratch_types=[
            pltpu.VMEM((BPW,), jnp.int32),
            pltpu.VMEM((BPW,), jnp.int32),
            pltpu.VMEM((BPW,), jnp.int32),
        ] + [pltpu.VMEM((CHUNK, D), jnp.float32)] * 3 + [
            pltpu.SemaphoreType.DMA,
        ],
    )
    def k(ia_hbm, ib_hbm, ir_hbm, ent_hbm, rel_hbm,
          a_out, b_out, r_out, av, bv, rv, b0, b1, b2, sem):
        wid = lax.axis_index("s") * NC + lax.axis_index("c")
        base = wid * BPW
        pltpu.sync_copy(ia_hbm.at[pl.ds(base, BPW)], av)
        pltpu.sync_copy(ib_hbm.at[pl.ds(base, BPW)], bv)
        pltpu.sync_copy(ir_hbm.at[pl.ds(base, BPW)], rv)
        for c in range(NCHUNK):
            cbase = c * CHUNK
            copies = [
                pltpu.async_copy(
                    ent_hbm.at[av.at[pl.ds(cbase, CHUNK)]], b0, sem),
                pltpu.async_copy(
                    ent_hbm.at[bv.at[pl.ds(cbase, CHUNK)]], b1, sem),
                pltpu.async_copy(
                    rel_hbm.at[rv.at[pl.ds(cbase, CHUNK)]], b2, sem),
            ]
            for cp in copies:
                cp.wait()
            for buf, out in zip((b0, b1, b2), (a_out, b_out, r_out)):
                pltpu.sync_copy(buf, out.at[pl.ds(base + cbase, CHUNK)])

    return k(idx_a, idx_b, idx_r, ent, rel)


def _tc_score(hh, ht, th, tt, r, rinv):
    blk = 2048

    def body(hh_ref, ht_ref, th_ref, tt_ref, r_ref, rinv_ref, o_ref):
        f = jnp.sum(hh_ref[...] * r_ref[...] * tt_ref[...], axis=1)
        inv = jnp.sum(ht_ref[...] * rinv_ref[...] * th_ref[...], axis=1)
        o_ref[...] = jnp.clip((f + inv) * 0.5, -20.0, 20.0)

    return pl.pallas_call(
        body,
        out_shape=jax.ShapeDtypeStruct((BATCH,), jnp.float32),
        grid=(BATCH // blk,),
        in_specs=[pl.BlockSpec((blk, D), lambda i: (i, 0))] * 6,
        out_specs=pl.BlockSpec((blk,), lambda i: (i,)),
    )(hh, ht, th, tt, r, rinv)


def kernel(heads, rels, tails, ent_h_embs, ent_t_embs, rel_embs, rel_inv_embs):
    heads = heads.astype(jnp.int32)
    rels = rels.astype(jnp.int32)
    tails = tails.astype(jnp.int32)
    hh, ht, r = _sc_gather3(heads, tails, rels, ent_h_embs, rel_embs)
    th, tt, rinv = _sc_gather3(heads, tails, rels, ent_t_embs, rel_inv_embs)
    return _tc_score(hh, ht, th, tt, r, rinv)


# speedup vs baseline: 1.5587x; 1.5587x over previous
import functools

import jax
import jax.numpy as jnp
from jax import lax
from jax.experimental import pallas as pl
from jax.experimental.pallas import tpu as pltpu
from jax.experimental.pallas import tpu_sc as plsc

BATCH = 16384
D = 64
NC, NS = 2, 16
NW = NC * NS
BPW = BATCH // NW
CHUNK = 128
NCHUNK = BPW // CHUNK


def _sc_gather_all(heads, rels, tails, ent_h, ent_t, rel, rel_inv):
    mesh = plsc.VectorSubcoreMesh(core_axis_name="c", subcore_axis_name="s")
    row_ty = jax.ShapeDtypeStruct((BATCH, D), jnp.float32)

    @functools.partial(
        pl.kernel,
        out_type=(row_ty,) * 6,
        mesh=mesh,
        scratch_types=[
            pltpu.VMEM((BPW,), jnp.int32),
            pltpu.VMEM((BPW,), jnp.int32),
            pltpu.VMEM((BPW,), jnp.int32),
        ] + [pltpu.VMEM((CHUNK, D), jnp.float32)] * 6 + [
            pltpu.SemaphoreType.DMA,
        ],
    )
    def k(heads_hbm, rels_hbm, tails_hbm, enth_hbm, entt_hbm, rel_hbm,
          relinv_hbm, hh_out, ht_out, th_out, tt_out, r_out, rinv_out,
          hv, rv, tv, b0, b1, b2, b3, b4, b5, sem):
        wid = lax.axis_index("s") * NC + lax.axis_index("c")
        base = wid * BPW
        pltpu.sync_copy(heads_hbm.at[pl.ds(base, BPW)], hv)
        pltpu.sync_copy(rels_hbm.at[pl.ds(base, BPW)], rv)
        pltpu.sync_copy(tails_hbm.at[pl.ds(base, BPW)], tv)
        bufs = (b0, b1, b2, b3, b4, b5)
        outs = (hh_out, ht_out, th_out, tt_out, r_out, rinv_out)
        for c in range(NCHUNK):
            cbase = c * CHUNK

            @pl.loop(0, CHUNK, step=16)
            def _(i):
                hvec = hv[pl.ds(cbase + i, 16)]
                tvec = tv[pl.ds(cbase + i, 16)]
                rvec = rv[pl.ds(cbase + i, 16)]
                for j in range(16):
                    h = hvec[j]
                    t = tvec[j]
                    r = rvec[j]
                    dst = pl.ds(i + j, 1)
                    pltpu.async_copy(
                        enth_hbm.at[pl.ds(h, 1)], b0.at[dst], sem)
                    pltpu.async_copy(
                        enth_hbm.at[pl.ds(t, 1)], b1.at[dst], sem)
                    pltpu.async_copy(
                        entt_hbm.at[pl.ds(h, 1)], b2.at[dst], sem)
                    pltpu.async_copy(
                        entt_hbm.at[pl.ds(t, 1)], b3.at[dst], sem)
                    pltpu.async_copy(
                        rel_hbm.at[pl.ds(r, 1)], b4.at[dst], sem)
                    pltpu.async_copy(
                        relinv_hbm.at[pl.ds(r, 1)], b5.at[dst], sem)

            for buf in bufs:
                pltpu.make_async_copy(
                    enth_hbm.at[pl.ds(0, CHUNK)], buf, sem).wait()
            for buf, out in zip(bufs, outs):
                pltpu.sync_copy(buf, out.at[pl.ds(base + cbase, CHUNK)])

    return k(heads, rels, tails, ent_h, ent_t, rel, rel_inv)


def _tc_score(hh, ht, th, tt, r, rinv):
    blk = 2048

    def body(hh_ref, ht_ref, th_ref, tt_ref, r_ref, rinv_ref, o_ref):
        f = jnp.sum(hh_ref[...] * r_ref[...] * tt_ref[...], axis=1)
        inv = jnp.sum(ht_ref[...] * rinv_ref[...] * th_ref[...], axis=1)
        o_ref[...] = jnp.clip((f + inv) * 0.5, -20.0, 20.0)

    return pl.pallas_call(
        body,
        out_shape=jax.ShapeDtypeStruct((BATCH,), jnp.float32),
        grid=(BATCH // blk,),
        in_specs=[pl.BlockSpec((blk, D), lambda i: (i, 0))] * 6,
        out_specs=pl.BlockSpec((blk,), lambda i: (i,)),
    )(hh, ht, th, tt, r, rinv)


def kernel(heads, rels, tails, ent_h_embs, ent_t_embs, rel_embs, rel_inv_embs):
    heads = heads.astype(jnp.int32)
    rels = rels.astype(jnp.int32)
    tails = tails.astype(jnp.int32)
    hh, ht, th, tt, r, rinv = _sc_gather_all(
        heads, rels, tails, ent_h_embs, ent_t_embs, rel_embs, rel_inv_embs)
    return _tc_score(hh, ht, th, tt, r, rinv)
